# baseline (speedup 1.0000x reference)
import functools

import jax
import jax.numpy as jnp
from jax.experimental import pallas as pl
from jax.experimental.pallas import tpu as pltpu

_VMEM_LIMIT = 48 * 1024 * 1024


def _scse_kernel(x_ref, w1t_ref, b1_ref, w2t_ref, b2_ref, ws_ref, bs_ref,
                 o_ref, cse_scr, *, hw, imgs, inv_hw, splits):
    h = pl.program_id(1)
    hw2 = hw // splits

    @pl.when(h == 0)
    def _():
        pools = []
        for i in range(imgs):
            pools.append(jnp.sum(x_ref[i], axis=0, keepdims=True))
        pooled = jnp.concatenate(pools, axis=0) * inv_hw
        z = jnp.dot(pooled, w1t_ref[...],
                    preferred_element_type=jnp.float32) + b1_ref[...]
        z = jnp.maximum(z, 0.0)
        s = jnp.dot(z, w2t_ref[...],
                    preferred_element_type=jnp.float32) + b2_ref[...]
        cse_scr[...] = jax.nn.sigmoid(s)

    xh = x_ref[:, pl.ds(h * hw2, hw2), :]
    x2 = xh.reshape(imgs * hw2, xh.shape[2])

    sp = jnp.dot(x2, ws_ref[...],
                 preferred_element_type=jnp.float32) + bs_ref[0]
    sse = jax.nn.sigmoid(sp).reshape(imgs, hw2, 1)

    o_ref[...] = xh * (cse_scr[...][:, None, :] + sse)


def kernel(x, w1, b1, w2, b2, ws, bs):
    N, C, H, W = x.shape
    HW = H * W
    mid = w1.shape[0]
    B = 2
    SPLITS = 2
    HW2 = HW // SPLITS

    xt = jnp.transpose(x, (0, 2, 3, 1)).reshape(N, HW, C)

    w1t = w1.astype(jnp.float32).T
    w2t = w2.astype(jnp.float32).T
    b1r = b1.reshape(1, mid).astype(jnp.float32)
    b2r = b2.reshape(1, C).astype(jnp.float32)
    ws_col = ws.reshape(1, C).T.astype(jnp.float32)
    bs_smem = bs.reshape(1).astype(jnp.float32)

    out = pl.pallas_call(
        functools.partial(_scse_kernel, hw=HW, imgs=B, inv_hw=1.0 / HW,
                          splits=SPLITS),
        out_shape=jax.ShapeDtypeStruct((N, HW, C), jnp.float32),
        grid_spec=pltpu.PrefetchScalarGridSpec(
            num_scalar_prefetch=0,
            grid=(N // B, SPLITS),
            in_specs=[
                pl.BlockSpec((B, HW, C), lambda n, h: (n, 0, 0)),
                pl.BlockSpec((C, mid), lambda n, h: (0, 0)),
                pl.BlockSpec((1, mid), lambda n, h: (0, 0)),
                pl.BlockSpec((mid, C), lambda n, h: (0, 0)),
                pl.BlockSpec((1, C), lambda n, h: (0, 0)),
                pl.BlockSpec((C, 1), lambda n, h: (0, 0)),
                pl.BlockSpec(memory_space=pltpu.MemorySpace.SMEM),
            ],
            out_specs=pl.BlockSpec((B, HW2, C), lambda n, h: (n, h, 0)),
            scratch_shapes=[pltpu.VMEM((B, C), jnp.float32)],
        ),
        compiler_params=pltpu.CompilerParams(
            dimension_semantics=("parallel", "arbitrary"),
            vmem_limit_bytes=_VMEM_LIMIT),
    )(xt, w1t, b1r, w2t, b2r, ws_col, bs_smem)

    return jnp.transpose(out.reshape(N, H, W, C), (0, 3, 1, 2))


# speedup vs baseline: 1.4104x; 1.4104x over previous
import functools

import jax
import jax.numpy as jnp
from jax.experimental import pallas as pl
from jax.experimental.pallas import tpu as pltpu

_VMEM_LIMIT = 48 * 1024 * 1024


def _scse_kernel(x_ref, w1t_ref, b1_ref, w2t_ref, b2_ref, ws_ref, bs_ref,
                 o_ref, *, hw, imgs, inv_hw):
    xf = x_ref[...]
    x2 = xf.reshape(imgs * hw, xf.shape[2])

    sp = jnp.dot(x2, ws_ref[...],
                 preferred_element_type=jnp.float32) + bs_ref[0]
    sse = jax.nn.sigmoid(sp).reshape(imgs, hw, 1)

    cses = []
    for i in range(imgs):
        pooled = jnp.sum(x2[i * hw:(i + 1) * hw], axis=0,
                         keepdims=True) * inv_hw
        z = jnp.dot(pooled, w1t_ref[...],
                    preferred_element_type=jnp.float32) + b1_ref[...]
        z = jnp.maximum(z, 0.0)
        s = jnp.dot(z, w2t_ref[...],
                    preferred_element_type=jnp.float32) + b2_ref[...]
        cses.append(jax.nn.sigmoid(s))
    cse = jnp.concatenate(cses, axis=0)[:, None, :]

    o_ref[...] = xf * (cse + sse)


def kernel(x, w1, b1, w2, b2, ws, bs):
    N, C, H, W = x.shape
    HW = H * W
    mid = w1.shape[0]
    B = 2

    xt = jnp.transpose(x, (0, 2, 3, 1)).reshape(N, HW, C)

    w1t = w1.astype(jnp.float32).T
    w2t = w2.astype(jnp.float32).T
    b1r = b1.reshape(1, mid).astype(jnp.float32)
    b2r = b2.reshape(1, C).astype(jnp.float32)
    ws_col = ws.reshape(1, C).T.astype(jnp.float32)
    bs_smem = bs.reshape(1).astype(jnp.float32)

    out = pl.pallas_call(
        functools.partial(_scse_kernel, hw=HW, imgs=B, inv_hw=1.0 / HW),
        out_shape=jax.ShapeDtypeStruct((N, HW, C), jnp.float32),
        grid_spec=pltpu.PrefetchScalarGridSpec(
            num_scalar_prefetch=0,
            grid=(N // B,),
            in_specs=[
                pl.BlockSpec((B, HW, C), lambda n: (n, 0, 0)),
                pl.BlockSpec((C, mid), lambda n: (0, 0)),
                pl.BlockSpec((1, mid), lambda n: (0, 0)),
                pl.BlockSpec((mid, C), lambda n: (0, 0)),
                pl.BlockSpec((1, C), lambda n: (0, 0)),
                pl.BlockSpec((C, 1), lambda n: (0, 0)),
                pl.BlockSpec(memory_space=pltpu.MemorySpace.SMEM),
            ],
            out_specs=pl.BlockSpec((B, HW, C), lambda n: (n, 0, 0)),
        ),
        compiler_params=pltpu.CompilerParams(
            dimension_semantics=("parallel",),
            vmem_limit_bytes=_VMEM_LIMIT),
    )(xt, w1t, b1r, w2t, b2r, ws_col, bs_smem)

    return jnp.transpose(out.reshape(N, H, W, C), (0, 3, 1, 2))


# speedup vs baseline: 1.4147x; 1.0031x over previous
import functools

import jax
import jax.numpy as jnp
from jax.experimental import pallas as pl
from jax.experimental.pallas import tpu as pltpu

_VMEM_LIMIT = 48 * 1024 * 1024


def _scse_kernel(x_ref, w1t_ref, b1_ref, w2t_ref, b2_ref, ws_ref, bs_ref,
                 o_ref, *, hw, imgs, inv_hw):
    xf = x_ref[...]
    x2 = xf.reshape(imgs * hw, xf.shape[2])

    sp = jnp.dot(x2, ws_ref[...],
                 preferred_element_type=jnp.float32) + bs_ref[0]
    sse = jax.nn.sigmoid(sp).reshape(imgs, hw, 1)

    cses = []
    for i in range(imgs):
        pooled = jnp.sum(x2[i * hw:(i + 1) * hw], axis=0,
                         keepdims=True) * inv_hw
        z = jnp.dot(pooled, w1t_ref[...],
                    preferred_element_type=jnp.float32) + b1_ref[...]
        z = jnp.maximum(z, 0.0)
        s = jnp.dot(z, w2t_ref[...],
                    preferred_element_type=jnp.float32) + b2_ref[...]
        cses.append(jax.nn.sigmoid(s))
    cse = jnp.concatenate(cses, axis=0)[:, None, :]

    o_ref[...] = xf * (cse + sse)


def kernel(x, w1, b1, w2, b2, ws, bs):
    N, C, H, W = x.shape
    HW = H * W
    mid = w1.shape[0]
    B = 2

    xt = jnp.transpose(x, (0, 2, 3, 1)).reshape(N, HW, C)

    w1t = w1.astype(jnp.float32).T
    w2t = w2.astype(jnp.float32).T
    b1r = b1.reshape(1, mid).astype(jnp.float32)
    b2r = b2.reshape(1, C).astype(jnp.float32)
    ws_col = ws.reshape(1, C).T.astype(jnp.float32)
    bs_smem = bs.reshape(1).astype(jnp.float32)

    out = pl.pallas_call(
        functools.partial(_scse_kernel, hw=HW, imgs=B, inv_hw=1.0 / HW),
        out_shape=jax.ShapeDtypeStruct((N, HW, C), jnp.float32),
        grid_spec=pltpu.PrefetchScalarGridSpec(
            num_scalar_prefetch=0,
            grid=(N // B,),
            in_specs=[
                pl.BlockSpec((B, HW, C), lambda n: (n, 0, 0)),
                pl.BlockSpec((C, mid), lambda n: (0, 0)),
                pl.BlockSpec((1, mid), lambda n: (0, 0)),
                pl.BlockSpec((mid, C), lambda n: (0, 0)),
                pl.BlockSpec((1, C), lambda n: (0, 0)),
                pl.BlockSpec((C, 1), lambda n: (0, 0)),
                pl.BlockSpec(memory_space=pltpu.MemorySpace.SMEM),
            ],
            out_specs=pl.BlockSpec((B, HW, C), lambda n: (n, 0, 0)),
        ),
        compiler_params=pltpu.CompilerParams(
            dimension_semantics=("arbitrary",),
            vmem_limit_bytes=_VMEM_LIMIT),
    )(xt, w1t, b1r, w2t, b2r, ws_col, bs_smem)

    return jnp.transpose(out.reshape(N, H, W, C), (0, 3, 1, 2))


# speedup vs baseline: 1.4589x; 1.0312x over previous
import functools

import jax
import jax.numpy as jnp
from jax.experimental import pallas as pl
from jax.experimental.pallas import tpu as pltpu

_VMEM_LIMIT = 48 * 1024 * 1024


def _scse_kernel(x0_ref, x1_ref, w1t_ref, b1_ref, w2t_ref, b2_ref, ws_ref,
                 bs_ref, o_ref, *, inv_hw):
    for i, x_ref in enumerate((x0_ref, x1_ref)):
        xf = x_ref[0]

        pooled = jnp.sum(xf, axis=0, keepdims=True) * inv_hw
        z = jnp.dot(pooled, w1t_ref[...],
                    preferred_element_type=jnp.float32) + b1_ref[...]
        z = jnp.maximum(z, 0.0)
        s = jnp.dot(z, w2t_ref[...],
                    preferred_element_type=jnp.float32) + b2_ref[...]
        cse = jax.nn.sigmoid(s)

        sp = jnp.dot(xf, ws_ref[...],
                     preferred_element_type=jnp.float32) + bs_ref[0]
        sse = jax.nn.sigmoid(sp)

        o_ref[i] = xf * (cse + sse)


def kernel(x, w1, b1, w2, b2, ws, bs):
    N, C, H, W = x.shape
    HW = H * W
    mid = w1.shape[0]
    B = 2

    xt = jnp.transpose(x, (0, 2, 3, 1)).reshape(N, HW, C)

    w1t = w1.astype(jnp.float32).T
    w2t = w2.astype(jnp.float32).T
    b1r = b1.reshape(1, mid).astype(jnp.float32)
    b2r = b2.reshape(1, C).astype(jnp.float32)
    ws_col = ws.reshape(1, C).T.astype(jnp.float32)
    bs_smem = bs.reshape(1).astype(jnp.float32)

    out = pl.pallas_call(
        functools.partial(_scse_kernel, inv_hw=1.0 / HW),
        out_shape=jax.ShapeDtypeStruct((N, HW, C), jnp.float32),
        grid_spec=pltpu.PrefetchScalarGridSpec(
            num_scalar_prefetch=0,
            grid=(N // B,),
            in_specs=[
                pl.BlockSpec((1, HW, C), lambda n: (2 * n, 0, 0)),
                pl.BlockSpec((1, HW, C), lambda n: (2 * n + 1, 0, 0)),
                pl.BlockSpec((C, mid), lambda n: (0, 0)),
                pl.BlockSpec((1, mid), lambda n: (0, 0)),
                pl.BlockSpec((mid, C), lambda n: (0, 0)),
                pl.BlockSpec((1, C), lambda n: (0, 0)),
                pl.BlockSpec((C, 1), lambda n: (0, 0)),
                pl.BlockSpec(memory_space=pltpu.MemorySpace.SMEM),
            ],
            out_specs=pl.BlockSpec((B, HW, C), lambda n: (n, 0, 0)),
        ),
        compiler_params=pltpu.CompilerParams(
            dimension_semantics=("parallel",),
            vmem_limit_bytes=_VMEM_LIMIT),
    )(xt, xt, w1t, b1r, w2t, b2r, ws_col, bs_smem)

    return jnp.transpose(out.reshape(N, H, W, C), (0, 3, 1, 2))
